# baseline (speedup 1.0000x reference)
import functools

import jax
import jax.numpy as jnp
from jax import lax
from jax.experimental import pallas as pl
from jax.experimental.pallas import tpu as pltpu
from jax.experimental.pallas import tpu_sc as plsc

B = 4096
L = 50
D = 32
LANES = 16
NW = 32
BPW = B // NW
GCH = 12
NG = (L - 1) // GCH


def _embed_kernel_body(u_idx, i_idx, h_idx, u_tab, i_tab, h_tab,
                       out,
                       uidx_v, iidx_v, hidx_v, urows, irows, acc,
                       sem_ui, sem_h):
    cid = lax.axis_index("c")
    sid = lax.axis_index("s")
    wid = sid * 2 + cid
    base = wid * BPW

    pltpu.sync_copy(u_idx.at[pl.ds(base, BPW)], uidx_v)
    pltpu.sync_copy(i_idx.at[pl.ds(base, BPW)], iidx_v)
    pltpu.sync_copy(h_idx.at[:, pl.ds(base, BPW)], hidx_v)

    g_u = pltpu.async_copy(u_tab.at[uidx_v], urows, sem_ui)
    g_i = pltpu.async_copy(i_tab.at[iidx_v], irows, sem_ui)

    pltpu.async_copy(h_tab.at[hidx_v.at[0]], acc, sem_h).wait()
    pltpu.async_copy(h_tab.at[hidx_v.at[L - 1]], acc, sem_h, add=True)

    def fire_group(g):
        for k in range(GCH):
            pltpu.async_copy(h_tab.at[hidx_v.at[1 + g * GCH + k]],
                             acc, sem_h, add=True)

    def drain(n):
        for _ in range(n):
            pltpu.make_async_copy(h_tab.at[pl.ds(0, BPW)], acc,
                                  sem_h).wait()

    fire_group(0)

    def pipelined(g, carry):
        fire_group(g)
        drain(GCH)
        return carry

    lax.fori_loop(1, NG, pipelined, 0)
    drain(GCH + 1)

    scale = jnp.full((LANES,), 1.0 / L, jnp.float32)

    def scl(r, carry):
        acc[r, pl.ds(0, LANES)] = acc[r, pl.ds(0, LANES)] * scale
        acc[r, pl.ds(LANES, LANES)] = acc[r, pl.ds(LANES, LANES)] * scale
        return carry

    lax.fori_loop(0, BPW, scl, 0)

    g_u.wait()
    g_i.wait()
    pltpu.sync_copy(urows, out.at[pl.ds(base, BPW), 0])
    pltpu.sync_copy(irows, out.at[pl.ds(base, BPW), 1])
    pltpu.sync_copy(acc, out.at[pl.ds(base, BPW), 2])


@jax.jit
def kernel(user_idx, item_idx, hist_idx, user_table, item_table, hist_table):
    u_idx = user_idx.reshape(B).astype(jnp.int32)
    i_idx = item_idx.reshape(B).astype(jnp.int32)
    h_idx = hist_idx.astype(jnp.int32).T

    mesh = plsc.VectorSubcoreMesh(core_axis_name="c", subcore_axis_name="s")
    run = functools.partial(
        pl.kernel,
        out_type=jax.ShapeDtypeStruct((B, 3, D), jnp.float32),
        mesh=mesh,
        compiler_params=pltpu.CompilerParams(use_tc_tiling_on_sc=False),
        scratch_types=[
            pltpu.VMEM((BPW,), jnp.int32),
            pltpu.VMEM((BPW,), jnp.int32),
            pltpu.VMEM((L, BPW), jnp.int32),
            pltpu.VMEM((BPW, D), jnp.float32),
            pltpu.VMEM((BPW, D), jnp.float32),
            pltpu.VMEM((BPW, D), jnp.float32),
            pltpu.SemaphoreType.DMA,
            pltpu.SemaphoreType.DMA,
        ],
    )(_embed_kernel_body)

    return run(u_idx, i_idx, h_idx, user_table, item_table, hist_table)


# speedup vs baseline: 1.0043x; 1.0043x over previous
import functools

import jax
import jax.numpy as jnp
from jax import lax
from jax.experimental import pallas as pl
from jax.experimental.pallas import tpu as pltpu
from jax.experimental.pallas import tpu_sc as plsc

B = 4096
L = 50
D = 32
LANES = 16
NW = 32
BPW = B // NW
GCH = 12
NG = (L - 1) // GCH


def _embed_kernel_body(u_idx, i_idx, h_idx, u_tab, i_tab, h_tab,
                       out,
                       uidx_v, iidx_v, hidx_v, urows, irows, acc,
                       sem_ui, sem_h):
    cid = lax.axis_index("c")
    sid = lax.axis_index("s")
    wid = sid * 2 + cid
    base = wid * BPW

    pltpu.sync_copy(u_idx.at[pl.ds(base, BPW)], uidx_v)
    pltpu.sync_copy(i_idx.at[pl.ds(base, BPW)], iidx_v)
    pltpu.sync_copy(h_idx.at[pl.ds(wid * L, L)], hidx_v)

    g_u = pltpu.async_copy(u_tab.at[uidx_v], urows, sem_ui)
    g_i = pltpu.async_copy(i_tab.at[iidx_v], irows, sem_ui)

    pltpu.async_copy(h_tab.at[hidx_v.at[0]], acc, sem_h).wait()
    pltpu.async_copy(h_tab.at[hidx_v.at[L - 1]], acc, sem_h, add=True)

    def fire_group(g):
        for k in range(GCH):
            pltpu.async_copy(h_tab.at[hidx_v.at[1 + g * GCH + k]],
                             acc, sem_h, add=True)

    def drain(n):
        for _ in range(n):
            pltpu.make_async_copy(h_tab.at[pl.ds(0, BPW)], acc,
                                  sem_h).wait()

    fire_group(0)

    def pipelined(g, carry):
        fire_group(g)
        drain(GCH)
        return carry

    lax.fori_loop(1, NG, pipelined, 0)
    drain(GCH + 1)

    scale = jnp.full((LANES,), 1.0 / L, jnp.float32)

    def scl(r, carry):
        acc[r, pl.ds(0, LANES)] = acc[r, pl.ds(0, LANES)] * scale
        acc[r, pl.ds(LANES, LANES)] = acc[r, pl.ds(LANES, LANES)] * scale
        return carry

    lax.fori_loop(0, BPW, scl, 0)

    g_u.wait()
    g_i.wait()
    pltpu.sync_copy(urows, out.at[pl.ds(base, BPW), 0])
    pltpu.sync_copy(irows, out.at[pl.ds(base, BPW), 1])
    pltpu.sync_copy(acc, out.at[pl.ds(base, BPW), 2])


@jax.jit
def kernel(user_idx, item_idx, hist_idx, user_table, item_table, hist_table):
    u_idx = user_idx.reshape(B).astype(jnp.int32)
    i_idx = item_idx.reshape(B).astype(jnp.int32)
    h_idx = (hist_idx.astype(jnp.int32)
             .reshape(NW, BPW, L)
             .transpose(0, 2, 1)
             .reshape(NW * L, BPW))

    mesh = plsc.VectorSubcoreMesh(core_axis_name="c", subcore_axis_name="s")
    run = functools.partial(
        pl.kernel,
        out_type=jax.ShapeDtypeStruct((B, 3, D), jnp.float32),
        mesh=mesh,
        compiler_params=pltpu.CompilerParams(use_tc_tiling_on_sc=False),
        scratch_types=[
            pltpu.VMEM((BPW,), jnp.int32),
            pltpu.VMEM((BPW,), jnp.int32),
            pltpu.VMEM((L, BPW), jnp.int32),
            pltpu.VMEM((BPW, D), jnp.float32),
            pltpu.VMEM((BPW, D), jnp.float32),
            pltpu.VMEM((BPW, D), jnp.float32),
            pltpu.SemaphoreType.DMA,
            pltpu.SemaphoreType.DMA,
        ],
    )(_embed_kernel_body)

    return run(u_idx, i_idx, h_idx, user_table, item_table, hist_table)


# speedup vs baseline: 1.0175x; 1.0132x over previous
import functools

import jax
import jax.numpy as jnp
from jax import lax
from jax.experimental import pallas as pl
from jax.experimental.pallas import tpu as pltpu
from jax.experimental.pallas import tpu_sc as plsc

B = 4096
L = 50
D = 32
LANES = 16
NW = 32
BPW = B // NW
GCH = 12
NG = (L - 1) // GCH


def _embed_kernel_body(u_idx, i_idx, h_idx, u_tab, i_tab, h_tab,
                       out_u, out_i, out_h,
                       uidx_v, iidx_v, hidx_v, urows, irows, acc,
                       sem_ui, sem_h):
    cid = lax.axis_index("c")
    sid = lax.axis_index("s")
    wid = sid * 2 + cid
    base = wid * BPW

    pltpu.sync_copy(u_idx.at[pl.ds(base, BPW)], uidx_v)
    pltpu.sync_copy(i_idx.at[pl.ds(base, BPW)], iidx_v)
    pltpu.sync_copy(h_idx.at[pl.ds(wid * L, L)], hidx_v)

    g_u = pltpu.async_copy(u_tab.at[uidx_v], urows, sem_ui)
    g_i = pltpu.async_copy(i_tab.at[iidx_v], irows, sem_ui)

    pltpu.async_copy(h_tab.at[hidx_v.at[0]], acc, sem_h).wait()
    pltpu.async_copy(h_tab.at[hidx_v.at[L - 1]], acc, sem_h, add=True)

    def fire_group(g):
        for k in range(GCH):
            pltpu.async_copy(h_tab.at[hidx_v.at[1 + g * GCH + k]],
                             acc, sem_h, add=True)

    def drain(n):
        for _ in range(n):
            pltpu.make_async_copy(h_tab.at[pl.ds(0, BPW)], acc,
                                  sem_h).wait()

    fire_group(0)

    def pipelined(g, carry):
        fire_group(g)
        drain(GCH)
        return carry

    lax.fori_loop(1, NG, pipelined, 0)
    drain(GCH + 1)

    scale = jnp.full((LANES,), 1.0 / L, jnp.float32)

    def scl(r, carry):
        acc[r, pl.ds(0, LANES)] = acc[r, pl.ds(0, LANES)] * scale
        acc[r, pl.ds(LANES, LANES)] = acc[r, pl.ds(LANES, LANES)] * scale
        return carry

    lax.fori_loop(0, BPW, scl, 0)

    g_u.wait()
    g_i.wait()
    pltpu.sync_copy(urows, out_u.at[pl.ds(base, BPW)])
    pltpu.sync_copy(irows, out_i.at[pl.ds(base, BPW)])
    pltpu.sync_copy(acc, out_h.at[pl.ds(base, BPW)])


@jax.jit
def kernel(user_idx, item_idx, hist_idx, user_table, item_table, hist_table):
    u_idx = user_idx.reshape(B).astype(jnp.int32)
    i_idx = item_idx.reshape(B).astype(jnp.int32)
    h_idx = (hist_idx.astype(jnp.int32)
             .reshape(NW, BPW, L)
             .transpose(0, 2, 1)
             .reshape(NW * L, BPW))

    mesh = plsc.VectorSubcoreMesh(core_axis_name="c", subcore_axis_name="s")
    run = functools.partial(
        pl.kernel,
        out_type=[jax.ShapeDtypeStruct((B, D), jnp.float32),
                  jax.ShapeDtypeStruct((B, D), jnp.float32),
                  jax.ShapeDtypeStruct((B, D), jnp.float32)],
        mesh=mesh,
        compiler_params=pltpu.CompilerParams(use_tc_tiling_on_sc=False),
        scratch_types=[
            pltpu.VMEM((BPW,), jnp.int32),
            pltpu.VMEM((BPW,), jnp.int32),
            pltpu.VMEM((L, BPW), jnp.int32),
            pltpu.VMEM((BPW, D), jnp.float32),
            pltpu.VMEM((BPW, D), jnp.float32),
            pltpu.VMEM((BPW, D), jnp.float32),
            pltpu.SemaphoreType.DMA,
            pltpu.SemaphoreType.DMA,
        ],
    )(_embed_kernel_body)

    e_u, e_i, e_h = run(u_idx, i_idx, h_idx, user_table, item_table,
                        hist_table)
    return jnp.stack([e_u, e_i, e_h], axis=1)
